# Initial kernel scaffold; baseline (speedup 1.0000x reference)
import jax
import jax.numpy as jnp
from jax.experimental import pallas as pl


def kernel(pts, scene_sdf, bounds):
    raise NotImplementedError("write your pallas kernel here")


# speedup vs baseline: 1.0377x; 1.0377x over previous
import functools

import jax
import jax.numpy as jnp
from jax import lax
from jax.experimental import pallas as pl
from jax.experimental.pallas import tpu as pltpu
from jax.experimental.pallas import tpu_sc as plsc

_GRID = 256
_PITCH = 1.0 / _GRID
_NC = 2
_NS = 16
_NW = _NC * _NS
_L = 16
_T = 5248
_GCH = 128
_NSUB = _T // _GCH


@functools.lru_cache(maxsize=None)
def _build(n_tiles: int):
    b_per_w = n_tiles * _T

    mesh = plsc.VectorSubcoreMesh(core_axis_name="c", subcore_axis_name="s")

    @functools.partial(
        pl.kernel,
        mesh=mesh,
        out_type=jax.ShapeDtypeStruct((_NW * b_per_w,), jnp.float32),
        scratch_types=[
            pltpu.VMEM((_T,), jnp.float32),
            pltpu.VMEM((_T,), jnp.float32),
            pltpu.VMEM((_T,), jnp.float32),
            pltpu.VMEM((_T,), jnp.int32),
            pltpu.VMEM((_T,), jnp.float32),
            pltpu.VMEM((9, _L), jnp.float32),
            pltpu.SemaphoreType.DMA,
        ],
    )
    def k(xs_h, ys_h, zs_h, consts_h, sdf_h, out_h, xv, yv, zv, idxv, resv, cv, gsem):
        wid = lax.axis_index("s") * _NC + lax.axis_index("c")
        base = wid * b_per_w

        pltpu.sync_copy(consts_h, cv)
        lo0 = cv[0]
        lo1 = cv[1]
        lo2 = cv[2]
        lb0 = cv[3]
        lb1 = cv[4]
        lb2 = cv[5]
        ub0 = cv[6]
        ub1 = cv[7]
        ub2 = cv[8]
        scale = jnp.full((_L,), float(_GRID), jnp.float32)
        zero = jnp.zeros((_L,), jnp.int32)
        neg10 = jnp.full((_L,), -10.0, jnp.float32)

        for t in range(n_tiles):
            tbase = base + t * _T
            pltpu.sync_copy(xs_h.at[pl.ds(tbase, _T)], xv)
            pltpu.sync_copy(ys_h.at[pl.ds(tbase, _T)], yv)
            pltpu.sync_copy(zs_h.at[pl.ds(tbase, _T)], zv)

            def cbody(i, _):
                sl = pl.ds(i * _L, _L)
                x = xv[sl]
                y = yv[sl]
                z = zv[sl]
                ix = ((x - lo0) * scale).astype(jnp.int32)
                iy = ((y - lo1) * scale).astype(jnp.int32)
                iz = ((z - lo2) * scale).astype(jnp.int32)
                idx = ix * (_GRID * _GRID) + iy * _GRID + iz
                inb = (x > lb0) & (x < ub0)
                inb &= (y > lb1) & (y < ub1)
                inb &= (z > lb2) & (z < ub2)
                idxv[sl] = jnp.where(inb, idx, zero)
                return 0

            lax.fori_loop(0, _T // _L, cbody, 0)

            def gfire(j, _):
                sl = pl.ds(j * _GCH, _GCH)
                pltpu.async_copy(sdf_h.at[idxv.at[sl]], resv.at[sl], gsem)
                return 0

            lax.fori_loop(0, _NSUB, gfire, 0)
            pltpu.make_async_copy(sdf_h.at[pl.ds(0, _T)], resv, gsem).wait()

            def mbody(i, _):
                sl = pl.ds(i * _L, _L)
                oob = idxv[sl] == zero
                resv[sl] = jnp.where(oob, neg10, resv[sl])
                return 0

            lax.fori_loop(0, _T // _L, mbody, 0)

            pltpu.sync_copy(resv, out_h.at[pl.ds(tbase, _T)])

    return k


def kernel(pts, scene_sdf, bounds):
    n = pts.shape[0]
    n_tiles = -(-n // (_NW * _T))
    n_pad = _NW * n_tiles * _T
    ptsf = pts.astype(jnp.float32)
    if n_pad > n:
        ptsf = jnp.concatenate(
            [ptsf, jnp.full((n_pad - n, 3), 2.0, jnp.float32)], axis=0
        )
    pts_t = ptsf.T
    lo = bounds[0].astype(jnp.float32)
    hi = bounds[1].astype(jnp.float32)
    consts = jnp.concatenate([lo, lo + _PITCH, hi - _PITCH])
    consts = jnp.broadcast_to(consts[:, None], (9, _L))
    out = _build(n_tiles)(
        pts_t[0], pts_t[1], pts_t[2], consts, scene_sdf.astype(jnp.float32)
    )
    return out[:n]
